# Initial kernel scaffold; baseline (speedup 1.0000x reference)
import jax
import jax.numpy as jnp
from jax.experimental import pallas as pl


def kernel(x, abs_x, idx, points, Wq, Wk, Wv, Wnq, Wnk, Wnv1, Wnv2):
    raise NotImplementedError("write your pallas kernel here")


# speedup vs baseline: 5.2371x; 5.2371x over previous
import functools

import jax
import jax.numpy as jnp
from jax import lax
from jax.experimental import pallas as pl
from jax.experimental.pallas import tpu as pltpu
from jax.experimental.pallas import tpu_sc as plsc

_NB = 256


def _local_body(xt_ref, ab_ref, mall_ref, wv_ref, wn_ref,
                lf_ref, att_ref, nall_ref):
    xb = xt_ref[0]
    xs = xb[:64] + xb[64:]
    ab = ab_ref[0]
    qt = lax.dot_general(mall_ref[...], ab, (((1,), (0,)), ((), ())),
                         preferred_element_type=jnp.float32)
    nall_ref[0] = lax.dot_general(wn_ref[...], ab, (((1,), (0,)), ((), ())),
                                  preferred_element_type=jnp.float32)
    for g in range(4):
        qg = qt[64 * g:64 * g + 64]
        logits = jnp.sum(xs * qg[:, None, :], axis=0)
        m = jnp.max(logits, axis=0, keepdims=True)
        e = jnp.exp(logits - m)
        a = e / jnp.sum(e, axis=0, keepdims=True)
        att_ref[0, g] = a
        yg = jnp.sum(xb * a[None], axis=1)
        lf_ref[0, 24 * g:24 * g + 24] = lax.dot_general(
            wv_ref[24 * g:24 * g + 24, :], yg, (((1,), (0,)), ((), ())),
            preferred_element_type=jnp.float32)


def _local_call(xt, ab, mall, wv, wn):
    B, C, K, N = xt.shape
    nblk = pl.cdiv(N, _NB)
    return pl.pallas_call(
        _local_body,
        grid=(B, nblk),
        in_specs=[
            pl.BlockSpec((1, C, K, _NB), lambda b, j: (b, 0, 0, j)),
            pl.BlockSpec((1, 64, _NB), lambda b, j: (b, 0, j)),
            pl.BlockSpec((256, 64), lambda b, j: (0, 0)),
            pl.BlockSpec((96, 128), lambda b, j: (0, 0)),
            pl.BlockSpec((128, 64), lambda b, j: (0, 0)),
        ],
        out_specs=[
            pl.BlockSpec((1, 96, _NB), lambda b, j: (b, 0, j)),
            pl.BlockSpec((1, 4, K, _NB), lambda b, j: (b, 0, 0, j)),
            pl.BlockSpec((1, 128, _NB), lambda b, j: (b, 0, j)),
        ],
        out_shape=[
            jax.ShapeDtypeStruct((B, 96, N), jnp.float32),
            jax.ShapeDtypeStruct((B, 4, K, N), jnp.float32),
            jax.ShapeDtypeStruct((B, 128, N), jnp.float32),
        ],
    )(xt, ab, mall, wv, wn)


def _centrality_sc(att2, idxt):
    B, G, KN = att2.shape
    N = 10000
    npart = 4
    chunk = KN // npart
    mesh = plsc.VectorSubcoreMesh(core_axis_name="c", subcore_axis_name="s")

    @functools.partial(
        pl.kernel,
        mesh=mesh,
        compiler_params=pltpu.CompilerParams(needs_layout_passes=False),
        out_type=jax.ShapeDtypeStruct((B * G * npart * N,), jnp.float32),
        scratch_types=[
            pltpu.VMEM((chunk,), jnp.float32),
            pltpu.VMEM((chunk,), jnp.int32),
            pltpu.VMEM((N,), jnp.float32),
        ],
    )
    def run(att_hbm, idx_hbm, out_hbm, att_v, idx_v, cent_v):
        wid = lax.axis_index("s") * 2 + lax.axis_index("c")
        pair = wid // npart
        part = wid % npart
        b = pair // G

        def zero_body(i, _):
            cent_v[pl.ds(i * 16, 16)] = jnp.zeros((16,), jnp.float32)
            return 0
        lax.fori_loop(0, N // 16, zero_body, 0)

        pltpu.sync_copy(att_hbm.at[pl.ds(pair * KN + part * chunk, chunk)],
                        att_v)
        pltpu.sync_copy(idx_hbm.at[pl.ds(b * KN + part * chunk, chunk)],
                        idx_v)

        def acc_body(i, _):
            av = att_v[pl.ds(i * 16, 16)]
            iv = idx_v[pl.ds(i * 16, 16)]
            plsc.addupdate_scatter(cent_v, [iv], av)
            return 0
        lax.fori_loop(0, chunk // 16, acc_body, 0)

        pltpu.sync_copy(cent_v, out_hbm.at[pl.ds(wid * N, N)])

    out = run(att2.reshape(-1), idxt.reshape(-1))
    return out.reshape(B, G, npart, N)


def _topk_body(centp_ref, nall_ref, tv_ref, nksel_ref, nvj_ref):
    G, P, N = centp_ref.shape[1:]
    K = 16
    cent = jnp.sum(centp_ref[0], axis=1)
    iota = lax.broadcasted_iota(jnp.int32, (G, N), 1)
    tvs, tis = [], []
    for _ in range(K):
        m = jnp.max(cent, axis=1, keepdims=True)
        am = jnp.min(jnp.where(cent == m, iota, N), axis=1, keepdims=True)
        cent = jnp.where(iota == am, -1.0, cent)
        tvs.append(m)
        tis.append(am)
    tv = jnp.concatenate(tvs, axis=1)
    ti = jnp.concatenate(tis, axis=1)
    tv_ref[0] = tv
    nall = nall_ref[0]
    iota_n = lax.broadcasted_iota(jnp.int32, (N, K), 0)
    for g in range(4):
        oh = (iota_n == ti[g:g + 1, :]).astype(jnp.float32)
        nksel_ref[0, 8 * g:8 * g + 8] = lax.dot_general(
            nall[32 + 8 * g:40 + 8 * g], oh, (((1,), (0,)), ((), ())),
            preferred_element_type=jnp.float32)
        nvj_ref[0, 8 * g:8 * g + 8] = lax.dot_general(
            nall[96 + 8 * g:104 + 8 * g], oh, (((1,), (0,)), ((), ())),
            preferred_element_type=jnp.float32)


def _topk_call(centp, nall):
    B, G, P, N = centp.shape
    K = 16
    return pl.pallas_call(
        _topk_body,
        grid=(B,),
        in_specs=[
            pl.BlockSpec((1, G, P, N), lambda b: (b, 0, 0, 0)),
            pl.BlockSpec((1, 128, N), lambda b: (b, 0, 0)),
        ],
        out_specs=[
            pl.BlockSpec((1, G, K), lambda b: (b, 0, 0)),
            pl.BlockSpec((1, 32, K), lambda b: (b, 0, 0)),
            pl.BlockSpec((1, 32, K), lambda b: (b, 0, 0)),
        ],
        out_shape=[
            jax.ShapeDtypeStruct((B, G, K), jnp.float32),
            jax.ShapeDtypeStruct((B, 32, K), jnp.float32),
            jax.ShapeDtypeStruct((B, 32, K), jnp.float32),
        ],
    )(centp, nall)


def _nl_body(nall_ref, tv_ref, nksel_ref, nvj_ref, nl_ref):
    nall = nall_ref[0]
    for g in range(4):
        nq_g = nall[8 * g:8 * g + 8]
        nvi_g = nall[64 + 8 * g:72 + 8 * g]
        nvij_g = nall[96 + 8 * g:104 + 8 * g]
        ks = nksel_ref[0, 8 * g:8 * g + 8]
        logits = lax.dot_general(ks, nq_g, (((0,), (0,)), ((), ())),
                                 preferred_element_type=jnp.float32)
        m = jnp.max(logits, axis=0, keepdims=True)
        e = jnp.exp(logits - m)
        attn = e / jnp.sum(e, axis=0, keepdims=True)
        tvr = tv_ref[0, g:g + 1, :]
        rows9 = jnp.concatenate(
            [nvj_ref[0, 8 * g:8 * g + 8] * tvr, tvr], axis=0)
        ts = lax.dot_general(rows9, attn, (((1,), (0,)), ((), ())),
                             preferred_element_type=jnp.float32)
        nl_ref[0, 8 * g:8 * g + 8] = nvi_g + ts[:8] - nvij_g * ts[8:9]


def _nl_call(nall, tv, nksel, nvj):
    B, _, N = nall.shape
    K = 16
    nblk = pl.cdiv(N, _NB)
    return pl.pallas_call(
        _nl_body,
        grid=(B, nblk),
        in_specs=[
            pl.BlockSpec((1, 128, _NB), lambda b, j: (b, 0, j)),
            pl.BlockSpec((1, 4, K), lambda b, j: (b, 0, 0)),
            pl.BlockSpec((1, 32, K), lambda b, j: (b, 0, 0)),
            pl.BlockSpec((1, 32, K), lambda b, j: (b, 0, 0)),
        ],
        out_specs=pl.BlockSpec((1, 32, _NB), lambda b, j: (b, 0, j)),
        out_shape=jax.ShapeDtypeStruct((B, 32, N), jnp.float32),
    )(nall, tv, nksel, nvj)


def kernel(x, abs_x, idx, points, Wq, Wk, Wv, Wnq, Wnk, Wnv1, Wnv2):
    B, C, N, K = x.shape
    G = 4
    xt = jnp.transpose(x, (0, 1, 3, 2))
    ab = abs_x[..., 0]
    mall = jnp.concatenate(
        [Wk[24 * g:24 * g + 24].T @ Wq[24 * g:24 * g + 24] for g in range(G)],
        axis=0)
    wn = jnp.concatenate([Wnq, Wnk, Wnv1, Wnv2], axis=0)

    lf, att, nall = _local_call(xt, ab, mall, Wv, wn)

    att2 = att.reshape(B, G, K * N)
    idxt = jnp.transpose(idx[:, 0], (0, 2, 1)).reshape(B, K * N)
    idxt = idxt.astype(jnp.int32)
    centp = _centrality_sc(att2, idxt)

    tv, nksel, nvj = _topk_call(centp, nall)
    nl = _nl_call(nall, tv, nksel, nvj)

    return jnp.concatenate([lf, nl], axis=1).reshape(B, 128, N, 1)
